# baseline (speedup 1.0000x reference)
import functools

import jax
import jax.numpy as jnp
from jax import lax
from jax.experimental import pallas as pl
from jax.experimental.pallas import tpu as pltpu
from jax.experimental.pallas import tpu_sc as plsc

_REPEATS = 4
_TOTAL_LENGTH = 65536
_EMBED_DIM = 32
_NUM_IDX = 16384


@jax.jit
def kernel(x, idx):
    info = plsc.get_sparse_core_info()
    nw = info.num_cores * info.num_subcores
    b_per_w = _NUM_IDX // nw
    mesh = plsc.VectorSubcoreMesh(core_axis_name="c", subcore_axis_name="s")

    @functools.partial(
        pl.kernel,
        mesh=mesh,
        out_type=jax.ShapeDtypeStruct((_EMBED_DIM, _TOTAL_LENGTH), jnp.float32),
        scratch_types=[
            pltpu.VMEM((b_per_w,), jnp.int32),
            pltpu.VMEM((_EMBED_DIM, b_per_w), jnp.float32),
        ]
        + [pltpu.SemaphoreType.DMA] * 8
        + [pltpu.SemaphoreType.DMA],
        compiler_params=pltpu.CompilerParams(use_tc_tiling_on_sc=False),
    )
    def gather_repeat(xt_hbm, idx_hbm, out_hbm, idx_v, col_v, *sems):
        gsems, wsem = sems[:8], sems[8]
        wid = lax.axis_index("s") * info.num_cores + lax.axis_index("c")
        base = wid * b_per_w
        pltpu.sync_copy(idx_hbm.at[pl.ds(base, b_per_w)], idx_v)
        gathers = [
            pltpu.make_async_copy(xt_hbm.at[j].at[idx_v], col_v.at[j], gsems[j % 8])
            for j in range(_EMBED_DIM)
        ]
        for g in gathers:
            g.start()
        for g in gathers:
            g.wait()
        writes = [
            pltpu.make_async_copy(
                col_v,
                out_hbm.at[:, pl.ds(r * _NUM_IDX + base, b_per_w)],
                wsem,
            )
            for r in range(_REPEATS)
        ]
        for w in writes:
            w.start()
        for w in writes:
            w.wait()

    return gather_repeat(x.T, idx).T


# speedup vs baseline: 4.7044x; 4.7044x over previous
import functools

import jax
import jax.numpy as jnp
from jax import lax
from jax.experimental import pallas as pl
from jax.experimental.pallas import tpu as pltpu
from jax.experimental.pallas import tpu_sc as plsc

_REPEATS = 4
_TOTAL_LENGTH = 65536
_EMBED_DIM = 32
_NUM_IDX = 16384


@jax.jit
def kernel(x, idx):
    info = plsc.get_sparse_core_info()
    nw = info.num_cores * info.num_subcores
    b_per_w = _NUM_IDX // nw
    mesh = plsc.VectorSubcoreMesh(core_axis_name="c", subcore_axis_name="s")

    @functools.partial(
        pl.kernel,
        mesh=mesh,
        out_type=jax.ShapeDtypeStruct((_TOTAL_LENGTH, _EMBED_DIM), jnp.float32),
        scratch_types=[
            pltpu.VMEM((b_per_w,), jnp.int32),
            pltpu.VMEM((b_per_w, _EMBED_DIM), jnp.float32),
            pltpu.SemaphoreType.DMA,
            pltpu.SemaphoreType.DMA,
        ],
        compiler_params=pltpu.CompilerParams(use_tc_tiling_on_sc=False),
    )
    def gather_repeat(x_hbm, idx_hbm, out_hbm, idx_v, rows_v, gsem, wsem):
        wid = lax.axis_index("s") * info.num_cores + lax.axis_index("c")
        base = wid * b_per_w
        pltpu.sync_copy(idx_hbm.at[pl.ds(base, b_per_w)], idx_v)
        pltpu.make_async_copy(x_hbm.at[idx_v], rows_v, gsem).start()
        pltpu.make_async_copy(x_hbm.at[idx_v], rows_v, gsem).wait()
        writes = [
            pltpu.make_async_copy(
                rows_v, out_hbm.at[pl.ds(r * _NUM_IDX + base, b_per_w)], wsem
            )
            for r in range(_REPEATS)
        ]
        for w in writes:
            w.start()
        for w in writes:
            w.wait()

    return gather_repeat(x, idx)
